# baseline (speedup 1.0000x reference)
import functools

import jax
import jax.numpy as jnp
from jax import lax
from jax.experimental import pallas as pl
from jax.experimental.pallas import tpu as pltpu
from jax.experimental.pallas import tpu_sc as plsc

_N = 10000
_E = 320000
_EPAD = 327680
_IDXROWS = _EPAD // 128
_NA = _N + 16
_RLO = 624
_RHI = 640
_CHUNK1 = 8
_CHUNK2 = 32
_CHUNK3 = 16

_MESH = plsc.VectorSubcoreMesh(core_axis_name="c", subcore_axis_name="s")


def _zero_tile_rows(s, zrows, acc):
    @pl.when(s < 15)
    def _():
        off = pl.multiple_of(s * _RLO, 8)
        pltpu.sync_copy(zrows.at[pl.ds(0, _RLO)], acc.at[pl.ds(off, _RLO)])

    @pl.when(s == 15)
    def _():
        pltpu.sync_copy(zrows, acc.at[pl.ds(15 * _RLO, _RHI)])


def _writeback_tile_rows(s, base_out, acc, out):
    @pl.when(s < 15)
    def _():
        off = pl.multiple_of(s * _RLO, 8)
        pltpu.sync_copy(acc.at[pl.ds(off, _RLO)],
                        out.at[pl.ds(pl.multiple_of(base_out + off, 8), _RLO)])

    @pl.when(s == 15)
    def _():
        off = 15 * _RLO
        pltpu.sync_copy(acc.at[pl.ds(off, _RHI)],
                        out.at[pl.ds(pl.multiple_of(base_out + off, 8), _RHI)])


def _edge_chunk_loop(tab, srcr, dstr, acc, dacc, onesv, srcv, dstv, rows,
                     sems, base, n_chunks, chunk, lo=0):
    gsems, ssems = sems

    def fire_scatter(j):
        sp = [pltpu.async_copy(rows.at[pl.ds((j % 2) * 128, 128)],
                               acc.at[dstv.at[j]], ssems[j % 2], add=True)]
        if dacc is not None:
            sp.append(pltpu.async_copy(onesv, dacc.at[dstv.at[j]],
                                       ssems[j % 2], add=True))
        return sp

    def body(ch, carry):
        r0 = pl.multiple_of(base + ch * chunk, 8)
        pltpu.sync_copy(srcr.at[pl.ds(r0, chunk)], srcv)
        pltpu.sync_copy(dstr.at[pl.ds(r0, chunk)], dstv)
        cps = [None] * chunk
        sps = [None] * chunk
        for j in range(chunk):
            if j >= 2:
                for sp in sps[j - 2]:
                    sp.wait()
            cps[j] = pltpu.async_copy(tab.at[srcv.at[j]],
                                      rows.at[pl.ds((j % 2) * 128, 128)],
                                      gsems[j % 2])
            if j >= 1:
                cps[j - 1].wait()
                sps[j - 1] = fire_scatter(j - 1)
        cps[chunk - 1].wait()
        sps[chunk - 1] = fire_scatter(chunk - 1)
        for sp in sps[chunk - 2]:
            sp.wait()
        for sp in sps[chunk - 1]:
            sp.wait()
        return carry

    lax.fori_loop(lo, n_chunks, body, 0)


def _col_split_pass(c, s, tlo, thi, srcr, dstr, zrows, out, dacc, onesv,
                    srcv, dstv, rows, acc, sems, chunk):
    _zero_tile_rows(s, zrows, acc)
    plsc.subcore_barrier()
    rows_per_tile = _IDXROWS // 16
    base = s * rows_per_tile
    n_chunks = rows_per_tile // chunk

    half = n_chunks // 2

    @pl.when(c == 0)
    def _():
        if dacc is None:
            _edge_chunk_loop(tlo, srcr, dstr, acc, None, None, srcv, dstv,
                             rows, sems, base, n_chunks, chunk)
        else:
            _edge_chunk_loop(tlo, srcr, dstr, acc, dacc, onesv, srcv, dstv,
                             rows, sems, base, half, chunk)
            _edge_chunk_loop(tlo, srcr, dstr, acc, None, None, srcv, dstv,
                             rows, sems, base, n_chunks, chunk, lo=half)

    @pl.when(c == 1)
    def _():
        if dacc is None:
            _edge_chunk_loop(thi, srcr, dstr, acc, None, None, srcv, dstv,
                             rows, sems, base, n_chunks, chunk)
        else:
            _edge_chunk_loop(thi, srcr, dstr, acc, None, None, srcv, dstv,
                             rows, sems, base, half, chunk)
            _edge_chunk_loop(thi, srcr, dstr, acc, dacc, onesv, srcv, dstv,
                             rows, sems, base, n_chunks, chunk, lo=half)

    plsc.subcore_barrier()
    _writeback_tile_rows(s, c * _N, acc, out)


def _agg1_body(tlo, thi, srcr, dstr, zrows, onesh, zdeg, out, degout,
               srcv, dstv, rows, onesv, acc, dacc,
               gsem0, gsem1, ssem0, ssem1):
    c = lax.axis_index("c")
    s = lax.axis_index("s")
    _zero_tile_rows(s, zdeg, dacc)
    pltpu.sync_copy(onesh, onesv)
    _col_split_pass(c, s, tlo, thi, srcr, dstr, zrows, out, dacc, onesv,
                    srcv, dstv, rows, acc,
                    ((gsem0, gsem1), (ssem0, ssem1)), _CHUNK1)
    _writeback_tile_rows(s, c * _N, dacc, degout)


def _agg2_body(t0, t1, t2, t3, srcr, dstr, zrows, outa, outb,
               srcv, dstv, rows, acc, gsem0, gsem1, ssem0, ssem1):
    c = lax.axis_index("c")
    s = lax.axis_index("s")
    sems = ((gsem0, gsem1), (ssem0, ssem1))
    _col_split_pass(c, s, t0, t1, srcr, dstr, zrows, outa, None, None,
                    srcv, dstv, rows, acc, sems, _CHUNK2)
    _col_split_pass(c, s, t2, t3, srcr, dstr, zrows, outb, None, None,
                    srcv, dstv, rows, acc, sems, _CHUNK2)


def _agg_edge_split_body(tab, srcr, dstr, zrows, out,
                         srcv, dstv, rows, acc,
                         gsem0, gsem1, ssem0, ssem1):
    c = lax.axis_index("c")
    s = lax.axis_index("s")
    _zero_tile_rows(s, zrows, acc)
    plsc.subcore_barrier()
    rows_per_tile = _IDXROWS // 32
    base = c * (_IDXROWS // 2) + s * rows_per_tile
    _edge_chunk_loop(tab, srcr, dstr, acc, None, None, srcv, dstv, rows,
                     ((gsem0, gsem1), (ssem0, ssem1)), base,
                     rows_per_tile // _CHUNK3, _CHUNK3)
    plsc.subcore_barrier()
    _writeback_tile_rows(s, c * _N, acc, out)


def _make_agg1():
    return pl.kernel(
        _agg1_body,
        out_type=(jax.ShapeDtypeStruct((2 * _N, 64), jnp.float32),
                  jax.ShapeDtypeStruct((2 * _N, 8), jnp.float32)),
        mesh=_MESH,
        scratch_types=[
            pltpu.VMEM((_CHUNK1, 128), jnp.int32),
            pltpu.VMEM((_CHUNK1, 128), jnp.int32),
            pltpu.VMEM((256, 64), jnp.float32),
            pltpu.VMEM((128, 8), jnp.float32),
            pltpu.VMEM_SHARED((_NA, 64), jnp.float32),
            pltpu.VMEM_SHARED((_NA, 8), jnp.float32),
            pltpu.SemaphoreType.DMA,
            pltpu.SemaphoreType.DMA,
            pltpu.SemaphoreType.DMA,
            pltpu.SemaphoreType.DMA,
        ],
        compiler_params=pltpu.CompilerParams(use_tc_tiling_on_sc=False),
    )


def _make_agg2():
    return pl.kernel(
        _agg2_body,
        out_type=(jax.ShapeDtypeStruct((2 * _N, 64), jnp.float32),
                  jax.ShapeDtypeStruct((2 * _N, 64), jnp.float32)),
        mesh=_MESH,
        scratch_types=[
            pltpu.VMEM((_CHUNK2, 128), jnp.int32),
            pltpu.VMEM((_CHUNK2, 128), jnp.int32),
            pltpu.VMEM((256, 64), jnp.float32),
            pltpu.VMEM_SHARED((_NA, 64), jnp.float32),
            pltpu.SemaphoreType.DMA,
            pltpu.SemaphoreType.DMA,
            pltpu.SemaphoreType.DMA,
            pltpu.SemaphoreType.DMA,
        ],
        compiler_params=pltpu.CompilerParams(use_tc_tiling_on_sc=False),
    )


def _make_agg_edge_split():
    return pl.kernel(
        _agg_edge_split_body,
        out_type=jax.ShapeDtypeStruct((2 * _N, 64), jnp.float32),
        mesh=_MESH,
        scratch_types=[
            pltpu.VMEM((_CHUNK3, 128), jnp.int32),
            pltpu.VMEM((_CHUNK3, 128), jnp.int32),
            pltpu.VMEM((256, 64), jnp.float32),
            pltpu.VMEM_SHARED((_NA, 64), jnp.float32),
            pltpu.SemaphoreType.DMA,
            pltpu.SemaphoreType.DMA,
            pltpu.SemaphoreType.DMA,
            pltpu.SemaphoreType.DMA,
        ],
        compiler_params=pltpu.CompilerParams(use_tc_tiling_on_sc=False),
    )


_BLK = 1000
_GRID = _N // _BLK


def _inv_deg(d0, d1):
    return 1.0 / jnp.maximum(d0[:, :1] + d1[:, :1], 1.0)


def _f32dot(a, b):
    return jnp.dot(a, b, preferred_element_type=jnp.float32)


def _self_body(x, w, b, o):
    o[...] = _f32dot(x[...], w[...]) + b[...]


def _self4_body(t0, t1, t2, t3, w, b, o):
    h = b[...]
    for k, t in enumerate((t0, t1, t2, t3)):
        h = h + _f32dot(t[...], w[64 * k:64 * (k + 1), :])
    o[...] = h


def _comb1_body(s1, a0, a1, d0, d1, wn, o0, o1, o2, o3):
    inv = _inv_deg(d0[...], d1[...])
    h = (s1[...] + _f32dot(a0[...] * inv, wn[:64, :])
         + _f32dot(a1[...] * inv, wn[64:, :]))
    h = jnp.maximum(h, 0.0)
    o0[...] = h[:, 0:64]
    o1[...] = h[:, 64:128]
    o2[...] = h[:, 128:192]
    o3[...] = h[:, 192:256]


def _comb2_body(s2, p0, p1, p2, p3, d0, d1, wn, wn3, oh, oq):
    inv = _inv_deg(d0[...], d1[...])
    h = s2[...]
    for k, p in enumerate((p0, p1, p2, p3)):
        h = h + _f32dot(p[...] * inv, wn[64 * k:64 * (k + 1), :])
    h = jnp.maximum(h, 0.0)
    oh[...] = h
    oq[...] = _f32dot(h, wn3[...])


def _comb3_body(s3, a0, a1, d0, d1, o):
    o[...] = s3[...] + (a0[...] + a1[...]) * _inv_deg(d0[...], d1[...])


def _row_spec(w):
    return pl.BlockSpec((_BLK, w), lambda i: (i, 0))


def _row_spec_hi(w):
    return pl.BlockSpec((_BLK, w), lambda i: (i + _GRID, 0))


def _full_spec(shape):
    return pl.BlockSpec(shape, lambda i: tuple(0 for _ in shape))


def _tc_self(x, w, b):
    din, dout = w.shape
    return pl.pallas_call(
        _self_body,
        grid=(_GRID,),
        in_specs=[_row_spec(din), _full_spec((din, dout)),
                  _full_spec((1, dout))],
        out_specs=_row_spec(dout),
        out_shape=jax.ShapeDtypeStruct((_N, dout), jnp.float32),
    )(x, w, b)


def _tc_self4(ts, w, b):
    dout = w.shape[1]
    return pl.pallas_call(
        _self4_body,
        grid=(_GRID,),
        in_specs=[_row_spec(64)] * 4 + [_full_spec((256, dout)),
                                        _full_spec((1, dout))],
        out_specs=_row_spec(dout),
        out_shape=jax.ShapeDtypeStruct((_N, dout), jnp.float32),
    )(*ts, w, b)


def _tc_comb1(s1, agg, deg, wn):
    return pl.pallas_call(
        _comb1_body,
        grid=(_GRID,),
        in_specs=[_row_spec(256), _row_spec(64), _row_spec_hi(64),
                  _row_spec(8), _row_spec_hi(8), _full_spec((128, 256))],
        out_specs=[_row_spec(64)] * 4,
        out_shape=[jax.ShapeDtypeStruct((_N, 64), jnp.float32)] * 4,
    )(s1, agg, agg, deg, deg, wn)


def _tc_comb2(s2, agg2a, agg2b, deg, wn, wn3):
    return pl.pallas_call(
        _comb2_body,
        grid=(_GRID,),
        in_specs=[_row_spec(256),
                  _row_spec(64), _row_spec_hi(64),
                  _row_spec(64), _row_spec_hi(64),
                  _row_spec(8), _row_spec_hi(8),
                  _full_spec((256, 256)), _full_spec((256, 64))],
        out_specs=[_row_spec(256), _row_spec(64)],
        out_shape=[jax.ShapeDtypeStruct((_N, 256), jnp.float32),
                   jax.ShapeDtypeStruct((_N, 64), jnp.float32)],
    )(s2, agg2a, agg2a, agg2b, agg2b, deg, deg, wn, wn3)


def _tc_comb3(s3, agg3, deg):
    return pl.pallas_call(
        _comb3_body,
        grid=(_GRID,),
        in_specs=[_row_spec(64), _row_spec(64), _row_spec_hi(64),
                  _row_spec(8), _row_spec_hi(8)],
        out_specs=_row_spec(64),
        out_shape=jax.ShapeDtypeStruct((_N, 64), jnp.float32),
    )(s3, agg3, agg3, deg, deg)


def kernel(features, edge_index, Ws1, Wn1, b1, Ws2, Wn2, b2, Ws3, Wn3, b3):
    src = edge_index[0]
    dst = edge_index[1]
    pad = _EPAD - _E
    padi = jnp.arange(pad, dtype=jnp.int32)
    src_p = jnp.concatenate([src, padi]).reshape(_IDXROWS, 128)
    dst_p = jnp.concatenate([dst, _N + (padi % 16)]).reshape(_IDXROWS, 128)
    z64 = jnp.zeros((_RHI, 64), jnp.float32)
    z8 = jnp.zeros((_RHI, 8), jnp.float32)
    ones8 = jnp.ones((128, 8), jnp.float32)

    flo = features[:, :64]
    fhi = features[:, 64:]
    agg1, deg = _make_agg1()(flo, fhi, src_p, dst_p, z64, ones8, z8)
    s1 = _tc_self(features, Ws1, b1.reshape(1, -1))
    ts = _tc_comb1(s1, agg1, deg, Wn1)
    agg2a, agg2b = _make_agg2()(ts[0], ts[1], ts[2], ts[3], src_p, dst_p, z64)
    s2 = _tc_self4(ts, Ws2, b2.reshape(1, -1))
    h2, q = _tc_comb2(s2, agg2a, agg2b, deg, Wn2, Wn3)
    agg3 = _make_agg_edge_split()(q, src_p, dst_p, z64)
    s3 = _tc_self(h2, Ws3, b3.reshape(1, -1))
    return _tc_comb3(s3, agg3, deg)


# speedup vs baseline: 1.0234x; 1.0234x over previous
import functools

import jax
import jax.numpy as jnp
from jax import lax
from jax.experimental import pallas as pl
from jax.experimental.pallas import tpu as pltpu
from jax.experimental.pallas import tpu_sc as plsc

_N = 10000
_E = 320000
_EPAD = 327680
_IDXROWS = _EPAD // 128
_NA = _N + 16
_RLO = 624
_RHI = 640
_CHUNK1 = 16
_CHUNK2 = 32
_CHUNK3 = 16

_MESH = plsc.VectorSubcoreMesh(core_axis_name="c", subcore_axis_name="s")


def _zero_tile_rows(s, zrows, acc):
    @pl.when(s < 15)
    def _():
        off = pl.multiple_of(s * _RLO, 8)
        pltpu.sync_copy(zrows.at[pl.ds(0, _RLO)], acc.at[pl.ds(off, _RLO)])

    @pl.when(s == 15)
    def _():
        pltpu.sync_copy(zrows, acc.at[pl.ds(15 * _RLO, _RHI)])


def _writeback_tile_rows(s, base_out, acc, out):
    @pl.when(s < 15)
    def _():
        off = pl.multiple_of(s * _RLO, 8)
        pltpu.sync_copy(acc.at[pl.ds(off, _RLO)],
                        out.at[pl.ds(pl.multiple_of(base_out + off, 8), _RLO)])

    @pl.when(s == 15)
    def _():
        off = 15 * _RLO
        pltpu.sync_copy(acc.at[pl.ds(off, _RHI)],
                        out.at[pl.ds(pl.multiple_of(base_out + off, 8), _RHI)])


def _edge_chunk_loop(tab, srcr, dstr, acc, dacc, onesv, srcv, dstv, rows,
                     sems, base, n_chunks, chunk, lo=0):
    gsems, ssems = sems

    def fire_scatter(j):
        sp = [pltpu.async_copy(rows.at[pl.ds((j % 2) * 128, 128)],
                               acc.at[dstv.at[j]], ssems[j % 2], add=True)]
        if dacc is not None:
            sp.append(pltpu.async_copy(onesv, dacc.at[dstv.at[j]],
                                       ssems[j % 2], add=True))
        return sp

    def body(ch, carry):
        r0 = pl.multiple_of(base + ch * chunk, 8)
        pltpu.sync_copy(srcr.at[pl.ds(r0, chunk)], srcv)
        pltpu.sync_copy(dstr.at[pl.ds(r0, chunk)], dstv)
        cps = [None] * chunk
        sps = [None] * chunk
        for j in range(chunk):
            if j >= 2:
                for sp in sps[j - 2]:
                    sp.wait()
            cps[j] = pltpu.async_copy(tab.at[srcv.at[j]],
                                      rows.at[pl.ds((j % 2) * 128, 128)],
                                      gsems[j % 2])
            if j >= 1:
                cps[j - 1].wait()
                sps[j - 1] = fire_scatter(j - 1)
        cps[chunk - 1].wait()
        sps[chunk - 1] = fire_scatter(chunk - 1)
        for sp in sps[chunk - 2]:
            sp.wait()
        for sp in sps[chunk - 1]:
            sp.wait()
        return carry

    lax.fori_loop(lo, n_chunks, body, 0)


def _col_split_pass(c, s, tlo, thi, srcr, dstr, zrows, out, dacc, onesv,
                    srcv, dstv, rows, acc, sems, chunk):
    _zero_tile_rows(s, zrows, acc)
    plsc.subcore_barrier()
    rows_per_tile = _IDXROWS // 16
    base = s * rows_per_tile
    n_chunks = rows_per_tile // chunk

    half = n_chunks // 2

    @pl.when(c == 0)
    def _():
        if dacc is None:
            _edge_chunk_loop(tlo, srcr, dstr, acc, None, None, srcv, dstv,
                             rows, sems, base, n_chunks, chunk)
        else:
            _edge_chunk_loop(tlo, srcr, dstr, acc, dacc, onesv, srcv, dstv,
                             rows, sems, base, half, chunk)
            _edge_chunk_loop(tlo, srcr, dstr, acc, None, None, srcv, dstv,
                             rows, sems, base, n_chunks, chunk, lo=half)

    @pl.when(c == 1)
    def _():
        if dacc is None:
            _edge_chunk_loop(thi, srcr, dstr, acc, None, None, srcv, dstv,
                             rows, sems, base, n_chunks, chunk)
        else:
            _edge_chunk_loop(thi, srcr, dstr, acc, None, None, srcv, dstv,
                             rows, sems, base, half, chunk)
            _edge_chunk_loop(thi, srcr, dstr, acc, dacc, onesv, srcv, dstv,
                             rows, sems, base, n_chunks, chunk, lo=half)

    plsc.subcore_barrier()
    _writeback_tile_rows(s, c * _N, acc, out)


def _agg1_body(tlo, thi, srcr, dstr, zrows, onesh, zdeg, out, degout,
               srcv, dstv, rows, onesv, acc, dacc,
               gsem0, gsem1, ssem0, ssem1):
    c = lax.axis_index("c")
    s = lax.axis_index("s")
    _zero_tile_rows(s, zdeg, dacc)
    pltpu.sync_copy(onesh, onesv)
    _col_split_pass(c, s, tlo, thi, srcr, dstr, zrows, out, dacc, onesv,
                    srcv, dstv, rows, acc,
                    ((gsem0, gsem1), (ssem0, ssem1)), _CHUNK1)
    _writeback_tile_rows(s, c * _N, dacc, degout)


def _agg2_body(t0, t1, t2, t3, srcr, dstr, zrows, outa, outb,
               srcv, dstv, rows, acc, gsem0, gsem1, ssem0, ssem1):
    c = lax.axis_index("c")
    s = lax.axis_index("s")
    sems = ((gsem0, gsem1), (ssem0, ssem1))
    _col_split_pass(c, s, t0, t1, srcr, dstr, zrows, outa, None, None,
                    srcv, dstv, rows, acc, sems, _CHUNK2)
    _col_split_pass(c, s, t2, t3, srcr, dstr, zrows, outb, None, None,
                    srcv, dstv, rows, acc, sems, _CHUNK2)


def _agg_edge_split_body(tab, srcr, dstr, zrows, out,
                         srcv, dstv, rows, acc,
                         gsem0, gsem1, ssem0, ssem1):
    c = lax.axis_index("c")
    s = lax.axis_index("s")
    _zero_tile_rows(s, zrows, acc)
    plsc.subcore_barrier()
    rows_per_tile = _IDXROWS // 32
    base = c * (_IDXROWS // 2) + s * rows_per_tile
    _edge_chunk_loop(tab, srcr, dstr, acc, None, None, srcv, dstv, rows,
                     ((gsem0, gsem1), (ssem0, ssem1)), base,
                     rows_per_tile // _CHUNK3, _CHUNK3)
    plsc.subcore_barrier()
    _writeback_tile_rows(s, c * _N, acc, out)


def _make_agg1():
    return pl.kernel(
        _agg1_body,
        out_type=(jax.ShapeDtypeStruct((2 * _N, 64), jnp.float32),
                  jax.ShapeDtypeStruct((2 * _N, 4), jnp.float32)),
        mesh=_MESH,
        scratch_types=[
            pltpu.VMEM((_CHUNK1, 128), jnp.int32),
            pltpu.VMEM((_CHUNK1, 128), jnp.int32),
            pltpu.VMEM((256, 64), jnp.float32),
            pltpu.VMEM((128, 4), jnp.float32),
            pltpu.VMEM_SHARED((_NA, 64), jnp.float32),
            pltpu.VMEM_SHARED((_NA, 4), jnp.float32),
            pltpu.SemaphoreType.DMA,
            pltpu.SemaphoreType.DMA,
            pltpu.SemaphoreType.DMA,
            pltpu.SemaphoreType.DMA,
        ],
        compiler_params=pltpu.CompilerParams(use_tc_tiling_on_sc=False),
    )


def _make_agg2():
    return pl.kernel(
        _agg2_body,
        out_type=(jax.ShapeDtypeStruct((2 * _N, 64), jnp.float32),
                  jax.ShapeDtypeStruct((2 * _N, 64), jnp.float32)),
        mesh=_MESH,
        scratch_types=[
            pltpu.VMEM((_CHUNK2, 128), jnp.int32),
            pltpu.VMEM((_CHUNK2, 128), jnp.int32),
            pltpu.VMEM((256, 64), jnp.float32),
            pltpu.VMEM_SHARED((_NA, 64), jnp.float32),
            pltpu.SemaphoreType.DMA,
            pltpu.SemaphoreType.DMA,
            pltpu.SemaphoreType.DMA,
            pltpu.SemaphoreType.DMA,
        ],
        compiler_params=pltpu.CompilerParams(use_tc_tiling_on_sc=False),
    )


def _make_agg_edge_split():
    return pl.kernel(
        _agg_edge_split_body,
        out_type=jax.ShapeDtypeStruct((2 * _N, 64), jnp.float32),
        mesh=_MESH,
        scratch_types=[
            pltpu.VMEM((_CHUNK3, 128), jnp.int32),
            pltpu.VMEM((_CHUNK3, 128), jnp.int32),
            pltpu.VMEM((256, 64), jnp.float32),
            pltpu.VMEM_SHARED((_NA, 64), jnp.float32),
            pltpu.SemaphoreType.DMA,
            pltpu.SemaphoreType.DMA,
            pltpu.SemaphoreType.DMA,
            pltpu.SemaphoreType.DMA,
        ],
        compiler_params=pltpu.CompilerParams(use_tc_tiling_on_sc=False),
    )


_BLK = 1000
_GRID = _N // _BLK


def _inv_deg(d0, d1):
    return 1.0 / jnp.maximum(d0[:, :1] + d1[:, :1], 1.0)


def _f32dot(a, b):
    return jnp.dot(a, b, preferred_element_type=jnp.float32)


def _self_body(x, w, b, o):
    o[...] = _f32dot(x[...], w[...]) + b[...]


def _self4_body(t0, t1, t2, t3, w, b, o):
    h = b[...]
    for k, t in enumerate((t0, t1, t2, t3)):
        h = h + _f32dot(t[...], w[64 * k:64 * (k + 1), :])
    o[...] = h


def _comb1_body(s1, a0, a1, d0, d1, wn, o0, o1, o2, o3):
    inv = _inv_deg(d0[...], d1[...])
    h = (s1[...] + _f32dot(a0[...] * inv, wn[:64, :])
         + _f32dot(a1[...] * inv, wn[64:, :]))
    h = jnp.maximum(h, 0.0)
    o0[...] = h[:, 0:64]
    o1[...] = h[:, 64:128]
    o2[...] = h[:, 128:192]
    o3[...] = h[:, 192:256]


def _comb2_body(s2, p0, p1, p2, p3, d0, d1, wn, wn3, oh, oq):
    inv = _inv_deg(d0[...], d1[...])
    h = s2[...]
    for k, p in enumerate((p0, p1, p2, p3)):
        h = h + _f32dot(p[...] * inv, wn[64 * k:64 * (k + 1), :])
    h = jnp.maximum(h, 0.0)
    oh[...] = h
    oq[...] = _f32dot(h, wn3[...])


def _comb3_body(s3, a0, a1, d0, d1, o):
    o[...] = s3[...] + (a0[...] + a1[...]) * _inv_deg(d0[...], d1[...])


def _row_spec(w):
    return pl.BlockSpec((_BLK, w), lambda i: (i, 0))


def _row_spec_hi(w):
    return pl.BlockSpec((_BLK, w), lambda i: (i + _GRID, 0))


def _full_spec(shape):
    return pl.BlockSpec(shape, lambda i: tuple(0 for _ in shape))


def _tc_self(x, w, b):
    din, dout = w.shape
    return pl.pallas_call(
        _self_body,
        grid=(_GRID,),
        in_specs=[_row_spec(din), _full_spec((din, dout)),
                  _full_spec((1, dout))],
        out_specs=_row_spec(dout),
        out_shape=jax.ShapeDtypeStruct((_N, dout), jnp.float32),
    )(x, w, b)


def _tc_self4(ts, w, b):
    dout = w.shape[1]
    return pl.pallas_call(
        _self4_body,
        grid=(_GRID,),
        in_specs=[_row_spec(64)] * 4 + [_full_spec((256, dout)),
                                        _full_spec((1, dout))],
        out_specs=_row_spec(dout),
        out_shape=jax.ShapeDtypeStruct((_N, dout), jnp.float32),
    )(*ts, w, b)


def _tc_comb1(s1, agg, deg, wn):
    return pl.pallas_call(
        _comb1_body,
        grid=(_GRID,),
        in_specs=[_row_spec(256), _row_spec(64), _row_spec_hi(64),
                  _row_spec(4), _row_spec_hi(4), _full_spec((128, 256))],
        out_specs=[_row_spec(64)] * 4,
        out_shape=[jax.ShapeDtypeStruct((_N, 64), jnp.float32)] * 4,
    )(s1, agg, agg, deg, deg, wn)


def _tc_comb2(s2, agg2a, agg2b, deg, wn, wn3):
    return pl.pallas_call(
        _comb2_body,
        grid=(_GRID,),
        in_specs=[_row_spec(256),
                  _row_spec(64), _row_spec_hi(64),
                  _row_spec(64), _row_spec_hi(64),
                  _row_spec(4), _row_spec_hi(4),
                  _full_spec((256, 256)), _full_spec((256, 64))],
        out_specs=[_row_spec(256), _row_spec(64)],
        out_shape=[jax.ShapeDtypeStruct((_N, 256), jnp.float32),
                   jax.ShapeDtypeStruct((_N, 64), jnp.float32)],
    )(s2, agg2a, agg2a, agg2b, agg2b, deg, deg, wn, wn3)


def _tc_comb3(s3, agg3, deg):
    return pl.pallas_call(
        _comb3_body,
        grid=(_GRID,),
        in_specs=[_row_spec(64), _row_spec(64), _row_spec_hi(64),
                  _row_spec(4), _row_spec_hi(4)],
        out_specs=_row_spec(64),
        out_shape=jax.ShapeDtypeStruct((_N, 64), jnp.float32),
    )(s3, agg3, agg3, deg, deg)


def kernel(features, edge_index, Ws1, Wn1, b1, Ws2, Wn2, b2, Ws3, Wn3, b3):
    src = edge_index[0]
    dst = edge_index[1]
    pad = _EPAD - _E
    padi = jnp.arange(pad, dtype=jnp.int32)
    src_p = jnp.concatenate([src, padi]).reshape(_IDXROWS, 128)
    dst_p = jnp.concatenate([dst, _N + (padi % 16)]).reshape(_IDXROWS, 128)
    z64 = jnp.zeros((_RHI, 64), jnp.float32)
    z8 = jnp.zeros((_RHI, 4), jnp.float32)
    ones8 = jnp.ones((128, 4), jnp.float32)

    flo = features[:, :64]
    fhi = features[:, 64:]
    agg1, deg = _make_agg1()(flo, fhi, src_p, dst_p, z64, ones8, z8)
    s1 = _tc_self(features, Ws1, b1.reshape(1, -1))
    ts = _tc_comb1(s1, agg1, deg, Wn1)
    agg2a, agg2b = _make_agg2()(ts[0], ts[1], ts[2], ts[3], src_p, dst_p, z64)
    s2 = _tc_self4(ts, Ws2, b2.reshape(1, -1))
    h2, q = _tc_comb2(s2, agg2a, agg2b, deg, Wn2, Wn3)
    agg3 = _make_agg_edge_split()(q, src_p, dst_p, z64)
    s3 = _tc_self(h2, Ws3, b3.reshape(1, -1))
    return _tc_comb3(s3, agg3, deg)
